# Initial kernel scaffold; baseline (speedup 1.0000x reference)
import jax
import jax.numpy as jnp
from jax.experimental import pallas as pl


def kernel(pos, edge_index_list, We1, be1, We2, be2, We3, be3, eln_g, eln_b, node_emb, ln_g, ln_b, phiW1, phib1, phiW2, phib2, thW, thb, Wf1, bf1, Wf2, bf2):
    raise NotImplementedError("write your pallas kernel here")


# speedup vs baseline: 1.9585x; 1.9585x over previous
import functools

import jax
import jax.numpy as jnp
from jax import lax
from jax.experimental import pallas as pl
from jax.experimental.pallas import tpu as pltpu
from jax.experimental.pallas import tpu_sc as plsc

_N = 10000
_E = 320000
_D = 128
_H = 128

_NC = 2
_NS = 16
_NW = _NC * _NS

_CB = 128
_NBLK = _E // _CB
_BLK_LO = _NBLK // _NW
_BLK_REM = _NBLK - _BLK_LO * _NW
_NP = 10240
_RPT = _NP // _NS

_BE = 2000
_GE = _E // _BE

_f32 = jnp.float32


def _mesh():
    return plsc.VectorSubcoreMesh(core_axis_name="c", subcore_axis_name="s",
                                  num_cores=_NC, num_subcores=_NS)


def _worker_id():
    return lax.axis_index("s") * _NC + lax.axis_index("c")


def _chunk_base(wid, i):
    return pl.multiple_of((wid + i * _NW) * _CB, _CB)


def _row_combine(dst_ref, a_ref, b_ref, op):
    def row(r, carry):
        for j in range(_D // 16):
            sl = pl.ds(j * 16, 16)
            dst_ref[r, sl] = op(a_ref[r, sl], b_ref[r, sl])
        return carry
    lax.fori_loop(0, _CB, row, 0, unroll=2)


def _make_sc_gather_combine(sub):
    @functools.partial(
        pl.kernel,
        out_type=jax.ShapeDtypeStruct((_E, _D), _f32),
        mesh=_mesh(),
        scratch_types=[
            pltpu.VMEM((_CB,), jnp.int32),
            pltpu.VMEM((_CB,), jnp.int32),
            pltpu.VMEM((_CB, _D), _f32),
            pltpu.VMEM((_CB, _D), _f32),
            pltpu.SemaphoreType.DMA,
            pltpu.SemaphoreType.DMA,
        ],
    )
    def k(tab, srci, dsti, out, ia, ib, ra, rb, s1, s2):
        wid = _worker_id()
        nblk = jnp.where(wid < _BLK_REM, _BLK_LO + 1, _BLK_LO)

        def body(i, carry):
            base = _chunk_base(wid, i)
            pltpu.sync_copy(srci.at[pl.ds(base, _CB)], ia)
            pltpu.sync_copy(dsti.at[pl.ds(base, _CB)], ib)
            c1 = pltpu.async_copy(tab.at[ia], ra, s1)
            c2 = pltpu.async_copy(tab.at[ib], rb, s2)
            c1.wait()
            c2.wait()
            if sub:
                _row_combine(rb, rb, ra, lambda b, a: b - a)
            else:
                _row_combine(rb, ra, rb, lambda a, b: a + b)
            pltpu.sync_copy(rb, out.at[pl.ds(base, _CB)])
            return carry

        lax.fori_loop(0, nblk, body, 0)

    return k


def _make_sc_scatter(mul):
    scratch = [
        pltpu.VMEM((_CB,), jnp.int32),
        pltpu.VMEM((_CB, _D), _f32),
        pltpu.VMEM_SHARED((_NP, _D), _f32),
        pltpu.SemaphoreType.DMA,
    ]
    if mul:
        scratch = scratch + [
            pltpu.VMEM((_CB,), jnp.int32),
            pltpu.VMEM((_CB, _D), _f32),
        ]

    @functools.partial(
        pl.kernel,
        out_type=jax.ShapeDtypeStruct((_NC, _NP, _D), _f32),
        mesh=_mesh(),
        scratch_types=scratch,
    )
    def k(*args):
        if mul:
            msg, srci, tab, dsti, out, isrc, tr, agg, s1, idst, nbr = args
        else:
            msg, srci, out, isrc, tr, agg, s1 = args
        cid = lax.axis_index("c")
        sid = lax.axis_index("s")
        wid = sid * _NC + cid
        nblk = jnp.where(wid < _BLK_REM, _BLK_LO + 1, _BLK_LO)

        def zrow(r, carry):
            for j in range(_D // 16):
                tr[r, pl.ds(j * 16, 16)] = jnp.zeros((16,), _f32)
            return carry
        lax.fori_loop(0, _CB, zrow, 0)
        row0 = sid * _RPT
        off = 0
        while off < _RPT:
            cnt = min(_CB, _RPT - off)
            pltpu.sync_copy(tr.at[pl.ds(0, cnt)], agg.at[pl.ds(row0 + off, cnt)])
            off += cnt
        plsc.subcore_barrier()

        def body(i, carry):
            base = _chunk_base(wid, i)
            pltpu.sync_copy(msg.at[pl.ds(base, _CB)], tr)
            pltpu.sync_copy(srci.at[pl.ds(base, _CB)], isrc)
            if mul:
                pltpu.sync_copy(dsti.at[pl.ds(base, _CB)], idst)
                pltpu.async_copy(tab.at[idst], nbr, s1).wait()
                _row_combine(tr, tr, nbr, lambda a, b: a * b)
            pltpu.sync_copy(tr, agg.at[isrc], add=True)
            return carry

        lax.fori_loop(0, nblk, body, 0)
        plsc.subcore_barrier()
        pltpu.sync_copy(agg.at[pl.ds(row0, _RPT)], out.at[cid, pl.ds(row0, _RPT)])

    return k


@functools.lru_cache(maxsize=None)
def _get_sc_kernels():
    return (_make_sc_gather_combine(sub=True),
            _make_sc_gather_combine(sub=False),
            _make_sc_scatter(mul=False),
            _make_sc_scatter(mul=True))


def _gelu(x):
    return 0.5 * x * (1.0 + lax.erf(x * 0.7071067811865476))


def _ln_rows(x, g, b):
    mu = jnp.mean(x, axis=-1, keepdims=True)
    var = jnp.mean((x - mu) ** 2, axis=-1, keepdims=True)
    return (x - mu) / jnp.sqrt(var + 1e-5) * g + b


def _dot(a, b):
    return jnp.dot(a, b, preferred_element_type=_f32)


def _tc_prep(pos, We1, emb, g0, b0):
    def body(p_ref, w_ref, e_ref, g_ref, b_ref, posw_ref, hn0_ref):
        p = p_ref[...]
        w = w_ref[...]
        posw_ref[...] = (p[:, 0:1] * w[0:1, :] + p[:, 1:2] * w[1:2, :]
                         + p[:, 2:3] * w[2:3, :])
        hn0_ref[...] = _ln_rows(e_ref[...], g_ref[...], b_ref[...])

    return pl.pallas_call(
        body,
        out_shape=[jax.ShapeDtypeStruct((_N, _D), _f32),
                   jax.ShapeDtypeStruct((1, _D), _f32)],
    )(pos, We1, emb, g0, b0)


def _edge_block_specs(n_edge_args):
    return [pl.BlockSpec((_BE, _D), lambda i: (i, 0)) for _ in range(n_edge_args)]


def _tc_edge_mlp(g, be1, We2, be2, We3, be3, eg, eb):
    def body(g_ref, b1, w2, b2, w3, b3, lg, lb, out_ref):
        h = _gelu(g_ref[...] + b1[...])
        h = _gelu(_dot(h, w2[...]) + b2[...])
        e = _dot(h, w3[...]) + b3[...]
        out_ref[...] = _ln_rows(e, lg[...], lb[...])

    wspec = pl.BlockSpec((_D, _D), lambda i: (0, 0))
    bspec = pl.BlockSpec((1, _D), lambda i: (0, 0))
    return pl.pallas_call(
        body,
        grid=(_GE,),
        in_specs=_edge_block_specs(1) + [bspec, wspec, bspec, wspec, bspec,
                                         bspec, bspec],
        out_specs=pl.BlockSpec((_BE, _D), lambda i: (i, 0)),
        out_shape=jax.ShapeDtypeStruct((_E, _D), _f32),
    )(g, be1, We2, be2, We3, be3, eg, eb)


def _tc_phi0(e, hn0, W1, b1, W2, b2):
    def body(e_ref, h0, w1, bb1, w2, bb2, out_ref):
        h0v = h0[...]
        u = jax.nn.silu(e_ref[...] + 2.0 * h0v)
        t = jax.nn.silu(_dot(u, w1[...]) + bb1[...])
        t = _dot(t, w2[...]) + bb2[...]
        out_ref[...] = t * h0v

    wspec = pl.BlockSpec((_D, _H), lambda i: (0, 0))
    bspec = pl.BlockSpec((1, _H), lambda i: (0, 0))
    wspec2 = pl.BlockSpec((_H, _D), lambda i: (0, 0))
    bspec2 = pl.BlockSpec((1, _D), lambda i: (0, 0))
    return pl.pallas_call(
        body,
        grid=(_GE,),
        in_specs=_edge_block_specs(1) + [bspec2, wspec, bspec, wspec2, bspec2],
        out_specs=pl.BlockSpec((_BE, _D), lambda i: (i, 0)),
        out_shape=jax.ShapeDtypeStruct((_E, _D), _f32),
    )(e, hn0, W1, b1, W2, b2)


def _tc_phi(e, s, W1, b1, W2, b2):
    def body(e_ref, s_ref, w1, bb1, w2, bb2, out_ref):
        u = jax.nn.silu(e_ref[...] + s_ref[...])
        t = jax.nn.silu(_dot(u, w1[...]) + bb1[...])
        out_ref[...] = _dot(t, w2[...]) + bb2[...]

    wspec = pl.BlockSpec((_D, _H), lambda i: (0, 0))
    bspec = pl.BlockSpec((1, _H), lambda i: (0, 0))
    wspec2 = pl.BlockSpec((_H, _D), lambda i: (0, 0))
    bspec2 = pl.BlockSpec((1, _D), lambda i: (0, 0))
    return pl.pallas_call(
        body,
        grid=(_GE,),
        in_specs=_edge_block_specs(2) + [wspec, bspec, wspec2, bspec2],
        out_specs=pl.BlockSpec((_BE, _D), lambda i: (i, 0)),
        out_shape=jax.ShapeDtypeStruct((_E, _D), _f32),
    )(e, s, W1, b1, W2, b2)


def _tc_update(node, hn, a0, a1, thW, thb, gn, bn):
    def body(n_ref, hn_ref, a0_ref, a1_ref, w_ref, b_ref, g_ref, lb_ref,
             out_ref, hnn_ref):
        x = jax.nn.silu(hn_ref[...] + a0_ref[...] + a1_ref[...])
        nn = _dot(x, w_ref[...]) + b_ref[...] + n_ref[...]
        out_ref[...] = nn
        hnn_ref[...] = _ln_rows(nn, g_ref[...], lb_ref[...])

    return pl.pallas_call(
        body,
        out_shape=[jax.ShapeDtypeStruct((_N, _D), _f32),
                   jax.ShapeDtypeStruct((_N, _D), _f32)],
    )(node, hn, a0, a1, thW, thb, gn, bn)


def _tc_update_final(node, hn, a0, a1, thW, thb, Wf1, bf1, Wf2p, bf2p):
    def body(n_ref, hn_ref, a0_ref, a1_ref, w_ref, b_ref, wf1, bff1, wf2,
             bff2, out_ref):
        x = jax.nn.silu(hn_ref[...] + a0_ref[...] + a1_ref[...])
        nn = _dot(x, w_ref[...]) + b_ref[...] + n_ref[...]
        f = _gelu(_dot(nn, wf1[...]) + bff1[...])
        out_ref[...] = _dot(f, wf2[...]) + bff2[...]

    return pl.pallas_call(
        body,
        out_shape=jax.ShapeDtypeStruct((_N, _D), _f32),
    )(node, hn, a0, a1, thW, thb, Wf1, bf1, Wf2p, bf2p)


def kernel(pos, edge_index_list, We1, be1, We2, be2, We3, be3, eln_g, eln_b,
           node_emb, ln_g, ln_b, phiW1, phib1, phiW2, phib2, thW, thb,
           Wf1, bf1, Wf2, bf2):
    src = edge_index_list[0]
    dst = edge_index_list[1]
    _sc_diff, _sc_sum, _sc_scatter_plain, _sc_scatter_mul = _get_sc_kernels()

    r1 = lambda v: v.reshape(1, -1)

    posW, hn0 = _tc_prep(pos, We1, node_emb, r1(ln_g[0]), r1(ln_b[0]))
    g = _sc_diff(posW, src, dst)
    e = _tc_edge_mlp(g, r1(be1), We2, r1(be2), We3, r1(be3), r1(eln_g),
                     r1(eln_b))

    msg0 = _tc_phi0(e, hn0, phiW1[0], r1(phib1[0]), phiW2[0], r1(phib2[0]))
    aggp = _sc_scatter_plain(msg0, src)
    a0, a1 = aggp[0, :_N], aggp[1, :_N]
    node0 = jnp.broadcast_to(node_emb, (_N, _D))
    hn0b = jnp.broadcast_to(hn0, (_N, _D))
    node, hn = _tc_update(node0, hn0b, a0, a1, thW[0], r1(thb[0]),
                          r1(ln_g[1]), r1(ln_b[1]))

    for l in (1, 2):
        s = _sc_sum(hn, src, dst)
        t = _tc_phi(e, s, phiW1[l], r1(phib1[l]), phiW2[l], r1(phib2[l]))
        aggp = _sc_scatter_mul(t, src, hn, dst)
        a0, a1 = aggp[0, :_N], aggp[1, :_N]
        node, hn = _tc_update(node, hn, a0, a1, thW[l], r1(thb[l]),
                              r1(ln_g[l + 1]), r1(ln_b[l + 1]))

    s = _sc_sum(hn, src, dst)
    t = _tc_phi(e, s, phiW1[3], r1(phib1[3]), phiW2[3], r1(phib2[3]))
    aggp = _sc_scatter_mul(t, src, hn, dst)
    a0, a1 = aggp[0, :_N], aggp[1, :_N]
    Wf2p = jnp.pad(Wf2, ((0, 0), (0, _D - Wf2.shape[1])))
    bf2p = jnp.pad(bf2, ((0, _D - bf2.shape[0]),)).reshape(1, _D)
    fpad = _tc_update_final(node, hn, a0, a1, thW[3], r1(thb[3]),
                            Wf1, r1(bf1), Wf2p, bf2p)
    return fpad[:, :3]


# speedup vs baseline: 4.4651x; 2.2799x over previous
import functools

import jax
import jax.numpy as jnp
from jax import lax
from jax.experimental import pallas as pl
from jax.experimental.pallas import tpu as pltpu
from jax.experimental.pallas import tpu_sc as plsc

_N = 10000
_E = 320000
_D = 128
_H = 128

_NC = 2
_NS = 16
_NW = _NC * _NS

_CB = 128
_NBLK = _E // _CB
_BLK_LO = _NBLK // _NW
_BLK_REM = _NBLK - _BLK_LO * _NW
_NP = 10112
_RPT = _NP // _NS

_BE = 2000
_GE = _E // _BE

_f32 = jnp.float32


def _mesh():
    return plsc.VectorSubcoreMesh(core_axis_name="c", subcore_axis_name="s",
                                  num_cores=_NC, num_subcores=_NS)


def _worker_id():
    return lax.axis_index("s") * _NC + lax.axis_index("c")


def _chunk_base(wid, i):
    return pl.multiple_of((wid + i * _NW) * _CB, _CB)


def _make_sc_gather2():
    nslot = 6
    scratch = []
    for _ in range(nslot):
        scratch += [pltpu.VMEM((_CB,), jnp.int32),
                    pltpu.VMEM((_CB, _D), _f32),
                    pltpu.SemaphoreType.DMA,
                    pltpu.SemaphoreType.DMA]

    @functools.partial(
        pl.kernel,
        out_type=[jax.ShapeDtypeStruct((_E, _D), _f32),
                  jax.ShapeDtypeStruct((_E, _D), _f32)],
        mesh=_mesh(),
        scratch_types=scratch,
    )
    def k(tab, srci, dsti, cout, nout, *scr):
        slots = [scr[i * 4:(i + 1) * 4] for i in range(nslot)]
        idxs = (srci, dsti)
        outs = (cout, nout)
        wid = _worker_id()

        def drain_out(s):
            ia, rows, sg, so = s
            pltpu.make_async_copy(rows, cout.at[pl.ds(0, _CB)], so).wait()

        def prefetch(chunk, which, s):
            ia, rows, sg, so = s
            base = _chunk_base(wid, chunk)
            pltpu.sync_copy(idxs[which].at[pl.ds(base, _CB)], ia)
            pltpu.async_copy(tab.at[ia], rows, sg)

        def use(chunk, which, s):
            ia, rows, sg, so = s
            pltpu.make_async_copy(tab.at[ia], rows, sg).wait()
            base = _chunk_base(wid, chunk)
            pltpu.async_copy(rows, outs[which].at[pl.ds(base, _CB)], so)

        for j in range(4):
            prefetch(j // 2, j % 2, slots[j])

        for b in range(6):
            use(b // 2, b % 2, slots[b])
            if b >= 2:
                drain_out(slots[(b + 4) % 6])
            prefetch((b + 4) // 2, b % 2, slots[(b + 4) % 6])

        def body(g, carry):
            for b in range(6):
                use(3 * g + b // 2, b % 2, slots[b])
                if b < 2:
                    drain_out(slots[(b + 4) % 6])
                    prefetch(3 * g + (b + 4) // 2, b % 2, slots[(b + 4) % 6])
                else:
                    @pl.when(g < 25)
                    def _():
                        drain_out(slots[(b + 4) % 6])
                        prefetch(3 * g + (b + 4) // 2, b % 2,
                                 slots[(b + 4) % 6])
            return carry

        lax.fori_loop(1, 26, body, 0)
        for b in range(nslot):
            drain_out(slots[b])

        @pl.when(wid < _BLK_REM)
        def _():
            ia, rows, sg, so = slots[0]
            base = _chunk_base(wid, _BLK_LO)
            for which in range(2):
                pltpu.sync_copy(idxs[which].at[pl.ds(base, _CB)], ia)
                pltpu.async_copy(tab.at[ia], rows, sg).wait()
                pltpu.sync_copy(rows, outs[which].at[pl.ds(base, _CB)])

    return k


def _make_sc_scatter():
    nslot = 3
    scratch = [pltpu.VMEM_SHARED((_NP, _D), _f32)]
    for _ in range(nslot):
        scratch += [pltpu.VMEM((_CB,), jnp.int32),
                    pltpu.VMEM((_CB, _D), _f32),
                    pltpu.SemaphoreType.DMA,
                    pltpu.SemaphoreType.DMA]

    @functools.partial(
        pl.kernel,
        out_type=jax.ShapeDtypeStruct((_NC, _NP, _D), _f32),
        mesh=_mesh(),
        scratch_types=scratch,
    )
    def k(msg, srci, out, agg, *scr):
        slots = [scr[i * 4:(i + 1) * 4] for i in range(nslot)]
        cid = lax.axis_index("c")
        sid = lax.axis_index("s")
        wid = sid * _NC + cid

        ia0, tr0, sl0, ss0 = slots[0]

        def zrow(r, carry):
            for j in range(_D // 16):
                tr0[r, pl.ds(j * 16, 16)] = jnp.zeros((16,), _f32)
            return carry

        lax.fori_loop(0, _CB, zrow, 0)
        row0 = sid * _RPT
        nfull = _RPT // _CB
        for off in range(0, nfull * _CB, _CB):
            pltpu.sync_copy(tr0.at[pl.ds(0, _CB)],
                            agg.at[pl.ds(row0 + off, _CB)])
        rem = _RPT - nfull * _CB
        if rem:
            pltpu.sync_copy(tr0.at[pl.ds(0, rem)],
                            agg.at[pl.ds(row0 + nfull * _CB, rem)])
        plsc.subcore_barrier()

        def drain_scatter(s):
            ia, tr, sl, ss = s
            pltpu.make_async_copy(tr, agg.at[ia], ss).wait()

        def prefetch(chunk, s):
            ia, tr, sl, ss = s
            base = _chunk_base(wid, chunk)
            pltpu.sync_copy(srci.at[pl.ds(base, _CB)], ia)
            pltpu.async_copy(msg.at[pl.ds(base, _CB)], tr, sl)

        def use(chunk, s):
            ia, tr, sl, ss = s
            base = _chunk_base(wid, chunk)
            pltpu.make_async_copy(msg.at[pl.ds(base, _CB)], tr, sl).wait()
            pltpu.async_copy(tr, agg.at[ia], ss, add=True)

        for j in range(2):
            prefetch(j, slots[j])

        for b in range(3):
            use(b, slots[b])
            if b >= 1:
                drain_scatter(slots[(b + 2) % 3])
            prefetch(b + 2, slots[(b + 2) % 3])

        def body(g, carry):
            for b in range(3):
                j = 3 * g + b
                use(j, slots[b])
                if b == 0:
                    drain_scatter(slots[(b + 2) % 3])
                    prefetch(j + 2, slots[(b + 2) % 3])
                else:
                    @pl.when(g < 25)
                    def _():
                        drain_scatter(slots[(b + 2) % 3])
                        prefetch(j + 2, slots[(b + 2) % 3])
            return carry

        lax.fori_loop(1, 26, body, 0)
        for b in range(nslot):
            drain_scatter(slots[b])

        @pl.when(wid < _BLK_REM)
        def _():
            ia, tr, sl, ss = slots[0]
            base = _chunk_base(wid, _BLK_LO)
            pltpu.sync_copy(srci.at[pl.ds(base, _CB)], ia)
            pltpu.sync_copy(msg.at[pl.ds(base, _CB)], tr)
            pltpu.sync_copy(tr, agg.at[ia], add=True)

        plsc.subcore_barrier()
        pltpu.sync_copy(agg.at[pl.ds(row0, _RPT)],
                        out.at[cid, pl.ds(row0, _RPT)])

    return k


@functools.lru_cache(maxsize=None)
def _get_sc_kernels():
    return (_make_sc_gather2(), _make_sc_scatter())


def _gelu(x):
    return 0.5 * x * (1.0 + lax.erf(x * 0.7071067811865476))


def _ln_rows(x, g, b):
    mu = jnp.mean(x, axis=-1, keepdims=True)
    var = jnp.mean((x - mu) ** 2, axis=-1, keepdims=True)
    return (x - mu) / jnp.sqrt(var + 1e-5) * g + b


def _dot(a, b):
    return jnp.dot(a, b, preferred_element_type=_f32)


def _tc_prep(pos, We1, emb, g0, b0):
    def body(p_ref, w_ref, e_ref, g_ref, b_ref, posw_ref, hn0_ref):
        p = p_ref[...]
        w = w_ref[...]
        posw_ref[...] = (p[:, 0:1] * w[0:1, :] + p[:, 1:2] * w[1:2, :]
                         + p[:, 2:3] * w[2:3, :])
        hn0_ref[...] = _ln_rows(e_ref[...], g_ref[...], b_ref[...])

    return pl.pallas_call(
        body,
        out_shape=[jax.ShapeDtypeStruct((_N, _D), _f32),
                   jax.ShapeDtypeStruct((1, _D), _f32)],
    )(pos, We1, emb, g0, b0)


def _edge_block_specs(n_edge_args):
    return [pl.BlockSpec((_BE, _D), lambda i: (i, 0)) for _ in range(n_edge_args)]


def _tc_edge_mlp(c0, n0, be1, We2, be2, We3, be3, eg, eb):
    def body(c_ref, n_ref, b1, w2, b2, w3, b3, lg, lb, out_ref):
        h = _gelu(n_ref[...] - c_ref[...] + b1[...])
        h = _gelu(_dot(h, w2[...]) + b2[...])
        e = _dot(h, w3[...]) + b3[...]
        out_ref[...] = _ln_rows(e, lg[...], lb[...])

    wspec = pl.BlockSpec((_D, _D), lambda i: (0, 0))
    bspec = pl.BlockSpec((1, _D), lambda i: (0, 0))
    return pl.pallas_call(
        body,
        grid=(_GE,),
        in_specs=_edge_block_specs(2) + [bspec, wspec, bspec, wspec, bspec,
                                         bspec, bspec],
        out_specs=pl.BlockSpec((_BE, _D), lambda i: (i, 0)),
        out_shape=jax.ShapeDtypeStruct((_E, _D), _f32),
    )(c0, n0, be1, We2, be2, We3, be3, eg, eb)


def _tc_phi0(e, hn0, W1, b1, W2, b2):
    def body(e_ref, h0, w1, bb1, w2, bb2, out_ref):
        h0v = h0[...]
        u = jax.nn.silu(e_ref[...] + 2.0 * h0v)
        t = jax.nn.silu(_dot(u, w1[...]) + bb1[...])
        t = _dot(t, w2[...]) + bb2[...]
        out_ref[...] = t * h0v

    wspec = pl.BlockSpec((_D, _H), lambda i: (0, 0))
    bspec = pl.BlockSpec((1, _H), lambda i: (0, 0))
    wspec2 = pl.BlockSpec((_H, _D), lambda i: (0, 0))
    bspec2 = pl.BlockSpec((1, _D), lambda i: (0, 0))
    return pl.pallas_call(
        body,
        grid=(_GE,),
        in_specs=_edge_block_specs(1) + [bspec2, wspec, bspec, wspec2, bspec2],
        out_specs=pl.BlockSpec((_BE, _D), lambda i: (i, 0)),
        out_shape=jax.ShapeDtypeStruct((_E, _D), _f32),
    )(e, hn0, W1, b1, W2, b2)


def _tc_phi(e, c, nb, W1, b1, W2, b2):
    def body(e_ref, c_ref, nb_ref, w1, bb1, w2, bb2, out_ref):
        nbv = nb_ref[...]
        u = jax.nn.silu(e_ref[...] + c_ref[...] + nbv)
        t = jax.nn.silu(_dot(u, w1[...]) + bb1[...])
        t = _dot(t, w2[...]) + bb2[...]
        out_ref[...] = t * nbv

    wspec = pl.BlockSpec((_D, _H), lambda i: (0, 0))
    bspec = pl.BlockSpec((1, _H), lambda i: (0, 0))
    wspec2 = pl.BlockSpec((_H, _D), lambda i: (0, 0))
    bspec2 = pl.BlockSpec((1, _D), lambda i: (0, 0))
    return pl.pallas_call(
        body,
        grid=(_GE,),
        in_specs=_edge_block_specs(3) + [wspec, bspec, wspec2, bspec2],
        out_specs=pl.BlockSpec((_BE, _D), lambda i: (i, 0)),
        out_shape=jax.ShapeDtypeStruct((_E, _D), _f32),
    )(e, c, nb, W1, b1, W2, b2)


def _tc_update(node, hn, a0, a1, thW, thb, gn, bn):
    def body(n_ref, hn_ref, a0_ref, a1_ref, w_ref, b_ref, g_ref, lb_ref,
             out_ref, hnn_ref):
        x = jax.nn.silu(hn_ref[...] + a0_ref[...] + a1_ref[...])
        nn = _dot(x, w_ref[...]) + b_ref[...] + n_ref[...]
        out_ref[...] = nn
        hnn_ref[...] = _ln_rows(nn, g_ref[...], lb_ref[...])

    return pl.pallas_call(
        body,
        out_shape=[jax.ShapeDtypeStruct((_N, _D), _f32),
                   jax.ShapeDtypeStruct((_N, _D), _f32)],
    )(node, hn, a0, a1, thW, thb, gn, bn)


def _tc_update_final(node, hn, a0, a1, thW, thb, Wf1, bf1, Wf2p, bf2p):
    def body(n_ref, hn_ref, a0_ref, a1_ref, w_ref, b_ref, wf1, bff1, wf2,
             bff2, out_ref):
        x = jax.nn.silu(hn_ref[...] + a0_ref[...] + a1_ref[...])
        nn = _dot(x, w_ref[...]) + b_ref[...] + n_ref[...]
        f = _gelu(_dot(nn, wf1[...]) + bff1[...])
        out_ref[...] = _dot(f, wf2[...]) + bff2[...]

    return pl.pallas_call(
        body,
        out_shape=jax.ShapeDtypeStruct((_N, _D), _f32),
    )(node, hn, a0, a1, thW, thb, Wf1, bf1, Wf2p, bf2p)


def kernel(pos, edge_index_list, We1, be1, We2, be2, We3, be3, eln_g, eln_b,
           node_emb, ln_g, ln_b, phiW1, phib1, phiW2, phib2, thW, thb,
           Wf1, bf1, Wf2, bf2):
    src = edge_index_list[0]
    dst = edge_index_list[1]
    _sc_gather2, _sc_scatter = _get_sc_kernels()

    r1 = lambda v: v.reshape(1, -1)

    posW, hn0 = _tc_prep(pos, We1, node_emb, r1(ln_g[0]), r1(ln_b[0]))
    c0, n0 = _sc_gather2(posW, src, dst)
    e = _tc_edge_mlp(c0, n0, r1(be1), We2, r1(be2), We3, r1(be3), r1(eln_g),
                     r1(eln_b))

    msg = _tc_phi0(e, hn0, phiW1[0], r1(phib1[0]), phiW2[0], r1(phib2[0]))
    aggp = _sc_scatter(msg, src)
    node0 = jnp.broadcast_to(node_emb, (_N, _D))
    hn0b = jnp.broadcast_to(hn0, (_N, _D))
    node, hn = _tc_update(node0, hn0b, aggp[0, :_N], aggp[1, :_N], thW[0],
                          r1(thb[0]), r1(ln_g[1]), r1(ln_b[1]))

    for l in (1, 2):
        c, nb = _sc_gather2(hn, src, dst)
        msg = _tc_phi(e, c, nb, phiW1[l], r1(phib1[l]), phiW2[l],
                      r1(phib2[l]))
        aggp = _sc_scatter(msg, src)
        node, hn = _tc_update(node, hn, aggp[0, :_N], aggp[1, :_N], thW[l],
                              r1(thb[l]), r1(ln_g[l + 1]), r1(ln_b[l + 1]))

    c, nb = _sc_gather2(hn, src, dst)
    msg = _tc_phi(e, c, nb, phiW1[3], r1(phib1[3]), phiW2[3], r1(phib2[3]))
    aggp = _sc_scatter(msg, src)
    Wf2p = jnp.pad(Wf2, ((0, 0), (0, _D - Wf2.shape[1])))
    bf2p = jnp.pad(bf2, ((0, _D - bf2.shape[0]),)).reshape(1, _D)
    fpad = _tc_update_final(node, hn, aggp[0, :_N], aggp[1, :_N], thW[3],
                            r1(thb[3]), Wf1, r1(bf1), Wf2p, bf2p)
    return fpad[:, :3]


# speedup vs baseline: 5.2139x; 1.1677x over previous
import functools

import jax
import jax.numpy as jnp
from jax import lax
from jax.experimental import pallas as pl
from jax.experimental.pallas import tpu as pltpu
from jax.experimental.pallas import tpu_sc as plsc

_N = 10000
_E = 320000
_D = 128
_H = 128

_NC = 2
_NS = 16
_NW = _NC * _NS

_CB = 128
_EH = _E // 2
_NBLK = _EH // _CB
_BLK_LO = _NBLK // _NW
_BLK_REM = _NBLK - _BLK_LO * _NW
_NGRP2 = (2 * _BLK_LO) // 6
_NGRP1 = _BLK_LO // 3
_NP = 10112
_RPT = _NP // _NS

_BE = 2000
_GE = _EH // _BE

_f32 = jnp.float32


def _mesh():
    return plsc.VectorSubcoreMesh(core_axis_name="c", subcore_axis_name="s",
                                  num_cores=_NC, num_subcores=_NS)


def _worker_id():
    return lax.axis_index("s") * _NC + lax.axis_index("c")


def _lbase(wid, i):
    return pl.multiple_of((wid + i * _NW) * _CB, _CB)


def _make_sc_gather2(half):
    goff = half * _EH
    nslot = 6
    scratch = []
    for _ in range(nslot):
        scratch += [pltpu.VMEM((_CB,), jnp.int32),
                    pltpu.VMEM((_CB, _D), _f32),
                    pltpu.SemaphoreType.DMA,
                    pltpu.SemaphoreType.DMA]

    @functools.partial(
        pl.kernel,
        out_type=[jax.ShapeDtypeStruct((_EH, _D), _f32),
                  jax.ShapeDtypeStruct((_EH, _D), _f32)],
        mesh=_mesh(),
        scratch_types=scratch,
    )
    def k(tab, srci, dsti, cout, nout, *scr):
        slots = [scr[i * 4:(i + 1) * 4] for i in range(nslot)]
        idxs = (srci, dsti)
        outs = (cout, nout)
        wid = _worker_id()

        def drain_out(s):
            ia, rows, sg, so = s
            pltpu.make_async_copy(rows, cout.at[pl.ds(0, _CB)], so).wait()

        def prefetch(chunk, which, s):
            ia, rows, sg, so = s
            base = _lbase(wid, chunk)
            pltpu.sync_copy(idxs[which].at[pl.ds(goff + base, _CB)], ia)
            pltpu.async_copy(tab.at[ia], rows, sg)

        def use(chunk, which, s):
            ia, rows, sg, so = s
            pltpu.make_async_copy(tab.at[ia], rows, sg).wait()
            base = _lbase(wid, chunk)
            pltpu.async_copy(rows, outs[which].at[pl.ds(base, _CB)], so)

        for j in range(4):
            prefetch(j // 2, j % 2, slots[j])

        for b in range(6):
            use(b // 2, b % 2, slots[b])
            if b >= 2:
                drain_out(slots[(b + 4) % 6])
            prefetch((b + 4) // 2, b % 2, slots[(b + 4) % 6])

        def body(g, carry):
            for b in range(6):
                use(3 * g + b // 2, b % 2, slots[b])
                if b < 2:
                    drain_out(slots[(b + 4) % 6])
                    prefetch(3 * g + (b + 4) // 2, b % 2, slots[(b + 4) % 6])
                else:
                    @pl.when(g < _NGRP2 - 1)
                    def _():
                        drain_out(slots[(b + 4) % 6])
                        prefetch(3 * g + (b + 4) // 2, b % 2,
                                 slots[(b + 4) % 6])
            return carry

        lax.fori_loop(1, _NGRP2, body, 0)
        for b in range(nslot):
            drain_out(slots[b])

        @pl.when(wid < _BLK_REM)
        def _():
            ia, rows, sg, so = slots[0]
            base = _lbase(wid, _BLK_LO)
            for which in range(2):
                pltpu.sync_copy(idxs[which].at[pl.ds(goff + base, _CB)], ia)
                pltpu.async_copy(tab.at[ia], rows, sg).wait()
                pltpu.sync_copy(rows, outs[which].at[pl.ds(base, _CB)])

    return k


def _make_sc_scatter(half):
    goff = half * _EH
    nslot = 3
    scratch = [pltpu.VMEM_SHARED((_NP, _D), _f32)]
    for _ in range(nslot):
        scratch += [pltpu.VMEM((_CB,), jnp.int32),
                    pltpu.VMEM((_CB, _D), _f32),
                    pltpu.SemaphoreType.DMA,
                    pltpu.SemaphoreType.DMA]

    @functools.partial(
        pl.kernel,
        out_type=jax.ShapeDtypeStruct((_NC, _NP, _D), _f32),
        mesh=_mesh(),
        scratch_types=scratch,
    )
    def k(msg, srci, out, agg, *scr):
        slots = [scr[i * 4:(i + 1) * 4] for i in range(nslot)]
        cid = lax.axis_index("c")
        sid = lax.axis_index("s")
        wid = sid * _NC + cid

        ia0, tr0, sl0, ss0 = slots[0]

        def zrow(r, carry):
            for j in range(_D // 16):
                tr0[r, pl.ds(j * 16, 16)] = jnp.zeros((16,), _f32)
            return carry

        lax.fori_loop(0, _CB, zrow, 0)
        row0 = sid * _RPT
        nfull = _RPT // _CB
        for off in range(0, nfull * _CB, _CB):
            pltpu.sync_copy(tr0.at[pl.ds(0, _CB)],
                            agg.at[pl.ds(row0 + off, _CB)])
        rem = _RPT - nfull * _CB
        if rem:
            pltpu.sync_copy(tr0.at[pl.ds(0, rem)],
                            agg.at[pl.ds(row0 + nfull * _CB, rem)])
        plsc.subcore_barrier()

        def drain_scatter(s):
            ia, tr, sl, ss = s
            pltpu.make_async_copy(tr, agg.at[ia], ss).wait()

        def prefetch(chunk, s):
            ia, tr, sl, ss = s
            base = _lbase(wid, chunk)
            pltpu.sync_copy(srci.at[pl.ds(goff + base, _CB)], ia)
            pltpu.async_copy(msg.at[pl.ds(base, _CB)], tr, sl)

        def use(chunk, s):
            ia, tr, sl, ss = s
            base = _lbase(wid, chunk)
            pltpu.make_async_copy(msg.at[pl.ds(base, _CB)], tr, sl).wait()
            pltpu.async_copy(tr, agg.at[ia], ss, add=True)

        for j in range(2):
            prefetch(j, slots[j])

        for b in range(3):
            use(b, slots[b])
            if b >= 1:
                drain_scatter(slots[(b + 2) % 3])
            prefetch(b + 2, slots[(b + 2) % 3])

        def body(g, carry):
            for b in range(3):
                j = 3 * g + b
                use(j, slots[b])
                if b == 0:
                    drain_scatter(slots[(b + 2) % 3])
                    prefetch(j + 2, slots[(b + 2) % 3])
                else:
                    @pl.when(g < _NGRP1 - 1)
                    def _():
                        drain_scatter(slots[(b + 2) % 3])
                        prefetch(j + 2, slots[(b + 2) % 3])
            return carry

        lax.fori_loop(1, _NGRP1, body, 0)
        for b in range(nslot):
            drain_scatter(slots[b])

        @pl.when(wid < _BLK_REM)
        def _():
            ia, tr, sl, ss = slots[0]
            base = _lbase(wid, _BLK_LO)
            pltpu.sync_copy(srci.at[pl.ds(goff + base, _CB)], ia)
            pltpu.sync_copy(msg.at[pl.ds(base, _CB)], tr)
            pltpu.sync_copy(tr, agg.at[ia], add=True)

        plsc.subcore_barrier()
        pltpu.sync_copy(agg.at[pl.ds(row0, _RPT)],
                        out.at[cid, pl.ds(row0, _RPT)])

    return k


@functools.lru_cache(maxsize=None)
def _get_sc_kernels():
    return ((_make_sc_gather2(0), _make_sc_gather2(1)),
            (_make_sc_scatter(0), _make_sc_scatter(1)))


def _gelu(x):
    return 0.5 * x * (1.0 + lax.erf(x * 0.7071067811865476))


def _ln_rows(x, g, b):
    mu = jnp.mean(x, axis=-1, keepdims=True)
    var = jnp.mean((x - mu) ** 2, axis=-1, keepdims=True)
    return (x - mu) / jnp.sqrt(var + 1e-5) * g + b


def _dot(a, b):
    return jnp.dot(a, b, preferred_element_type=_f32)


def _tc_prep(pos, We1, emb, g0, b0):
    def body(p_ref, w_ref, e_ref, g_ref, b_ref, posw_ref, hn0_ref):
        p = p_ref[...]
        w = w_ref[...]
        posw_ref[...] = (p[:, 0:1] * w[0:1, :] + p[:, 1:2] * w[1:2, :]
                         + p[:, 2:3] * w[2:3, :])
        hn0_ref[...] = _ln_rows(e_ref[...], g_ref[...], b_ref[...])

    return pl.pallas_call(
        body,
        out_shape=[jax.ShapeDtypeStruct((_N, _D), _f32),
                   jax.ShapeDtypeStruct((1, _D), _f32)],
    )(pos, We1, emb, g0, b0)


def _edge_block_specs(n_edge_args):
    return [pl.BlockSpec((_BE, _D), lambda i: (i, 0)) for _ in range(n_edge_args)]


def _tc_edge_mlp_phi0(c0, n0, be1, We2, be2, We3, be3, eg, eb, hn0,
                     W1, b1, W2, b2):
    def body(c_ref, n_ref, bb, w2, b2_, w3, b3_, lg, lb, h0, pw1, pb1, pw2,
             pb2, e_ref, m_ref):
        h = _gelu(n_ref[...] - c_ref[...] + bb[...])
        h = _gelu(_dot(h, w2[...]) + b2_[...])
        e = _dot(h, w3[...]) + b3_[...]
        e = _ln_rows(e, lg[...], lb[...])
        e_ref[...] = e
        h0v = h0[...]
        u = jax.nn.silu(e + 2.0 * h0v)
        t = jax.nn.silu(_dot(u, pw1[...]) + pb1[...])
        t = _dot(t, pw2[...]) + pb2[...]
        m_ref[...] = t * h0v

    wspec = pl.BlockSpec((_D, _D), lambda i: (0, 0))
    bspec = pl.BlockSpec((1, _D), lambda i: (0, 0))
    return pl.pallas_call(
        body,
        grid=(_GE,),
        in_specs=_edge_block_specs(2) + [bspec, wspec, bspec, wspec, bspec,
                                         bspec, bspec, bspec, wspec, bspec,
                                         wspec, bspec],
        out_specs=[pl.BlockSpec((_BE, _D), lambda i: (i, 0)),
                   pl.BlockSpec((_BE, _D), lambda i: (i, 0))],
        out_shape=[jax.ShapeDtypeStruct((_EH, _D), _f32),
                   jax.ShapeDtypeStruct((_EH, _D), _f32)],
    )(c0, n0, be1, We2, be2, We3, be3, eg, eb, hn0, W1, b1, W2, b2)


def _tc_phi(e, c, nb, W1, b1, W2, b2):
    def body(e_ref, c_ref, nb_ref, w1, bb1, w2, bb2, out_ref):
        nbv = nb_ref[...]
        u = jax.nn.silu(e_ref[...] + c_ref[...] + nbv)
        t = jax.nn.silu(_dot(u, w1[...]) + bb1[...])
        t = _dot(t, w2[...]) + bb2[...]
        out_ref[...] = t * nbv

    wspec = pl.BlockSpec((_D, _H), lambda i: (0, 0))
    bspec = pl.BlockSpec((1, _H), lambda i: (0, 0))
    wspec2 = pl.BlockSpec((_H, _D), lambda i: (0, 0))
    bspec2 = pl.BlockSpec((1, _D), lambda i: (0, 0))
    return pl.pallas_call(
        body,
        grid=(_GE,),
        in_specs=_edge_block_specs(3) + [wspec, bspec, wspec2, bspec2],
        out_specs=pl.BlockSpec((_BE, _D), lambda i: (i, 0)),
        out_shape=jax.ShapeDtypeStruct((_EH, _D), _f32),
    )(e, c, nb, W1, b1, W2, b2)


def _tc_update(node, hn, a00, a01, a10, a11, thW, thb, gn, bn):
    def body(n_ref, hn_ref, a00_ref, a01_ref, a10_ref, a11_ref, w_ref, b_ref,
             g_ref, lb_ref, out_ref, hnn_ref):
        agg = (a00_ref[...] + a01_ref[...]) + (a10_ref[...] + a11_ref[...])
        x = jax.nn.silu(hn_ref[...] + agg)
        nn = _dot(x, w_ref[...]) + b_ref[...] + n_ref[...]
        out_ref[...] = nn
        hnn_ref[...] = _ln_rows(nn, g_ref[...], lb_ref[...])

    return pl.pallas_call(
        body,
        out_shape=[jax.ShapeDtypeStruct((_N, _D), _f32),
                   jax.ShapeDtypeStruct((_N, _D), _f32)],
    )(node, hn, a00, a01, a10, a11, thW, thb, gn, bn)


def _tc_update_final(node, hn, a00, a01, a10, a11, thW, thb, Wf1, bf1, Wf2p,
                     bf2p):
    def body(n_ref, hn_ref, a00_ref, a01_ref, a10_ref, a11_ref, w_ref, b_ref,
             wf1, bff1, wf2, bff2, out_ref):
        agg = (a00_ref[...] + a01_ref[...]) + (a10_ref[...] + a11_ref[...])
        x = jax.nn.silu(hn_ref[...] + agg)
        nn = _dot(x, w_ref[...]) + b_ref[...] + n_ref[...]
        f = _gelu(_dot(nn, wf1[...]) + bff1[...])
        out_ref[...] = _dot(f, wf2[...]) + bff2[...]

    return pl.pallas_call(
        body,
        out_shape=jax.ShapeDtypeStruct((_N, _D), _f32),
    )(node, hn, a00, a01, a10, a11, thW, thb, Wf1, bf1, Wf2p, bf2p)


def kernel(pos, edge_index_list, We1, be1, We2, be2, We3, be3, eln_g, eln_b,
           node_emb, ln_g, ln_b, phiW1, phib1, phiW2, phib2, thW, thb,
           Wf1, bf1, Wf2, bf2):
    src = edge_index_list[0]
    dst = edge_index_list[1]
    g2, sct = _get_sc_kernels()

    r1 = lambda v: v.reshape(1, -1)

    posW, hn0 = _tc_prep(pos, We1, node_emb, r1(ln_g[0]), r1(ln_b[0]))
    eh = []
    aggs = []
    for h in (0, 1):
        c0, n0 = g2[h](posW, src, dst)
        e_h, msg = _tc_edge_mlp_phi0(c0, n0, r1(be1), We2, r1(be2), We3,
                                     r1(be3), r1(eln_g), r1(eln_b), hn0,
                                     phiW1[0], r1(phib1[0]), phiW2[0],
                                     r1(phib2[0]))
        eh.append(e_h)
        aggs.append(sct[h](msg, src))
    node0 = jnp.broadcast_to(node_emb, (_N, _D))
    hn0b = jnp.broadcast_to(hn0, (_N, _D))
    node, hn = _tc_update(node0, hn0b,
                          aggs[0][0, :_N], aggs[0][1, :_N],
                          aggs[1][0, :_N], aggs[1][1, :_N],
                          thW[0], r1(thb[0]), r1(ln_g[1]), r1(ln_b[1]))

    for l in (1, 2, 3):
        aggs = []
        for h in (0, 1):
            c, nb = g2[h](hn, src, dst)
            msg = _tc_phi(eh[h], c, nb, phiW1[l], r1(phib1[l]), phiW2[l],
                          r1(phib2[l]))
            aggs.append(sct[h](msg, src))
        if l < 3:
            node, hn = _tc_update(node, hn,
                                  aggs[0][0, :_N], aggs[0][1, :_N],
                                  aggs[1][0, :_N], aggs[1][1, :_N],
                                  thW[l], r1(thb[l]), r1(ln_g[l + 1]),
                                  r1(ln_b[l + 1]))

    Wf2p = jnp.pad(Wf2, ((0, 0), (0, _D - Wf2.shape[1])))
    bf2p = jnp.pad(bf2, ((0, _D - bf2.shape[0]),)).reshape(1, _D)
    fpad = _tc_update_final(node, hn,
                            aggs[0][0, :_N], aggs[0][1, :_N],
                            aggs[1][0, :_N], aggs[1][1, :_N],
                            thW[3], r1(thb[3]), Wf1, r1(bf1), Wf2p, bf2p)
    return fpad[:, :3]


# speedup vs baseline: 6.2511x; 1.1989x over previous
import functools

import jax
import jax.numpy as jnp
from jax import lax
from jax.experimental import pallas as pl
from jax.experimental.pallas import tpu as pltpu
from jax.experimental.pallas import tpu_sc as plsc

_N = 10000
_E = 320000
_D = 128
_H = 128

_NC = 2
_NS = 16
_NW = _NC * _NS

_CB = 128
_EH = _E // 2
_NBLK = _EH // _CB
_BLK_LO = _NBLK // _NW
_BLK_REM = _NBLK - _BLK_LO * _NW
_NGRP2 = (2 * _BLK_LO) // 6
_NGRP1 = _BLK_LO // 3
_NP = 10112
_RPT = _NP // _NS

_BE = 2000
_GE = _EH // _BE

_f32 = jnp.float32


def _mesh():
    return plsc.VectorSubcoreMesh(core_axis_name="c", subcore_axis_name="s",
                                  num_cores=_NC, num_subcores=_NS)


def _worker_id():
    return lax.axis_index("s") * _NC + lax.axis_index("c")


def _lbase(wid, i):
    return pl.multiple_of((wid + i * _NW) * _CB, _CB)


def _make_sc_gather2(half):
    goff = half * _EH
    nslot = 3
    scratch = [pltpu.VMEM_SHARED((_N, _D), _f32)]
    for _ in range(nslot):
        scratch += [pltpu.VMEM((_CB,), jnp.int32),
                    pltpu.VMEM((_CB, _D), _f32),
                    pltpu.SemaphoreType.DMA,
                    pltpu.SemaphoreType.DMA]

    @functools.partial(
        pl.kernel,
        out_type=[jax.ShapeDtypeStruct((_EH, _D), _f32),
                  jax.ShapeDtypeStruct((_EH, _D), _f32)],
        mesh=_mesh(),
        scratch_types=scratch,
    )
    def k(tab, srci, dsti, cout, nout, stab, *scr):
        slots = [scr[i * 4:(i + 1) * 4] for i in range(nslot)]
        idxs = (srci, dsti)
        outs = (cout, nout)
        wid = _worker_id()
        sid = lax.axis_index("s")

        srow = sid * 632

        @pl.when(sid < _NS - 1)
        def _():
            pltpu.sync_copy(tab.at[pl.ds(srow, 632)],
                            stab.at[pl.ds(srow, 632)])

        @pl.when(sid == _NS - 1)
        def _():
            pltpu.sync_copy(tab.at[pl.ds(srow, _N - (_NS - 1) * 632)],
                            stab.at[pl.ds(srow, _N - (_NS - 1) * 632)])
        plsc.subcore_barrier()

        def drain_out(s):
            ia, rows, sg, so = s
            pltpu.make_async_copy(rows, cout.at[pl.ds(0, _CB)], so).wait()

        def prefetch(chunk, which, s):
            ia, rows, sg, so = s
            base = _lbase(wid, chunk)
            pltpu.sync_copy(idxs[which].at[pl.ds(goff + base, _CB)], ia)
            pltpu.async_copy(stab.at[ia], rows, sg)

        def use(chunk, which, s):
            ia, rows, sg, so = s
            pltpu.make_async_copy(stab.at[ia], rows, sg).wait()
            base = _lbase(wid, chunk)
            pltpu.async_copy(rows, outs[which].at[pl.ds(base, _CB)], so)

        prefetch(0, 0, slots[0])
        prefetch(0, 1, slots[1])

        for b in range(6):
            use(b // 2, b % 2, slots[b % 3])
            if b > 0:
                drain_out(slots[(b + 2) % 3])
            prefetch((b + 2) // 2, b % 2, slots[(b + 2) % 3])

        def body(g, carry):
            for b in range(6):
                j = 6 * g + b
                use(3 * g + b // 2, b % 2, slots[b % 3])
                if b < 4:
                    drain_out(slots[(b + 2) % 3])
                    prefetch(3 * g + (b + 2) // 2, b % 2, slots[(b + 2) % 3])
                else:
                    @pl.when(g < 12)
                    def _():
                        drain_out(slots[(b + 2) % 3])
                        prefetch(3 * g + (b + 2) // 2, b % 2,
                                 slots[(b + 2) % 3])
            return carry

        lax.fori_loop(1, 13, body, 0)
        for b in range(nslot):
            drain_out(slots[b])

        @pl.when(wid < _BLK_REM)
        def _():
            ia, rows, sg, so = slots[0]
            base = _lbase(wid, _BLK_LO)
            for which in range(2):
                pltpu.sync_copy(idxs[which].at[pl.ds(goff + base, _CB)], ia)
                pltpu.async_copy(stab.at[ia], rows, sg).wait()
                pltpu.sync_copy(rows, outs[which].at[pl.ds(base, _CB)])

    return k


def _make_sc_scatter(half):
    goff = half * _EH
    nslot = 3
    scratch = [pltpu.VMEM_SHARED((_NP, _D), _f32)]
    for _ in range(nslot):
        scratch += [pltpu.VMEM((_CB,), jnp.int32),
                    pltpu.VMEM((_CB, _D), _f32),
                    pltpu.SemaphoreType.DMA,
                    pltpu.SemaphoreType.DMA]

    @functools.partial(
        pl.kernel,
        out_type=jax.ShapeDtypeStruct((_NC, _NP, _D), _f32),
        mesh=_mesh(),
        scratch_types=scratch,
    )
    def k(msg, srci, out, agg, *scr):
        slots = [scr[i * 4:(i + 1) * 4] for i in range(nslot)]
        cid = lax.axis_index("c")
        sid = lax.axis_index("s")
        wid = sid * _NC + cid

        ia0, tr0, sl0, ss0 = slots[0]

        def zrow(r, carry):
            for j in range(_D // 16):
                tr0[r, pl.ds(j * 16, 16)] = jnp.zeros((16,), _f32)
            return carry

        lax.fori_loop(0, _CB, zrow, 0)
        row0 = sid * _RPT
        nfull = _RPT // _CB
        for off in range(0, nfull * _CB, _CB):
            pltpu.sync_copy(tr0.at[pl.ds(0, _CB)],
                            agg.at[pl.ds(row0 + off, _CB)])
        rem = _RPT - nfull * _CB
        if rem:
            pltpu.sync_copy(tr0.at[pl.ds(0, rem)],
                            agg.at[pl.ds(row0 + nfull * _CB, rem)])
        plsc.subcore_barrier()

        def drain_scatter(s):
            ia, tr, sl, ss = s
            pltpu.make_async_copy(tr, agg.at[ia], ss).wait()

        def prefetch(chunk, s):
            ia, tr, sl, ss = s
            base = _lbase(wid, chunk)
            pltpu.sync_copy(srci.at[pl.ds(goff + base, _CB)], ia)
            pltpu.async_copy(msg.at[pl.ds(base, _CB)], tr, sl)

        def use(chunk, s):
            ia, tr, sl, ss = s
            base = _lbase(wid, chunk)
            pltpu.make_async_copy(msg.at[pl.ds(base, _CB)], tr, sl).wait()
            pltpu.async_copy(tr, agg.at[ia], ss, add=True)

        for j in range(2):
            prefetch(j, slots[j])

        for b in range(3):
            use(b, slots[b])
            if b >= 1:
                drain_scatter(slots[(b + 2) % 3])
            prefetch(b + 2, slots[(b + 2) % 3])

        def body(g, carry):
            for b in range(3):
                j = 3 * g + b
                use(j, slots[b])
                if b == 0:
                    drain_scatter(slots[(b + 2) % 3])
                    prefetch(j + 2, slots[(b + 2) % 3])
                else:
                    @pl.when(g < _NGRP1 - 1)
                    def _():
                        drain_scatter(slots[(b + 2) % 3])
                        prefetch(j + 2, slots[(b + 2) % 3])
            return carry

        lax.fori_loop(1, _NGRP1, body, 0)
        for b in range(nslot):
            drain_scatter(slots[b])

        @pl.when(wid < _BLK_REM)
        def _():
            ia, tr, sl, ss = slots[0]
            base = _lbase(wid, _BLK_LO)
            pltpu.sync_copy(srci.at[pl.ds(goff + base, _CB)], ia)
            pltpu.sync_copy(msg.at[pl.ds(base, _CB)], tr)
            pltpu.sync_copy(tr, agg.at[ia], add=True)

        plsc.subcore_barrier()
        pltpu.sync_copy(agg.at[pl.ds(row0, _RPT)],
                        out.at[cid, pl.ds(row0, _RPT)])

    return k


@functools.lru_cache(maxsize=None)
def _get_sc_kernels():
    return ((_make_sc_gather2(0), _make_sc_gather2(1)),
            (_make_sc_scatter(0), _make_sc_scatter(1)))


def _gelu(x):
    return 0.5 * x * (1.0 + lax.erf(x * 0.7071067811865476))


def _ln_rows(x, g, b):
    mu = jnp.mean(x, axis=-1, keepdims=True)
    var = jnp.mean((x - mu) ** 2, axis=-1, keepdims=True)
    return (x - mu) / jnp.sqrt(var + 1e-5) * g + b


def _dot(a, b):
    return jnp.dot(a, b, preferred_element_type=_f32)


def _tc_prep(pos, We1, emb, g0, b0):
    def body(p_ref, w_ref, e_ref, g_ref, b_ref, posw_ref, hn0_ref):
        p = p_ref[...]
        w = w_ref[...]
        posw_ref[...] = (p[:, 0:1] * w[0:1, :] + p[:, 1:2] * w[1:2, :]
                         + p[:, 2:3] * w[2:3, :])
        hn0_ref[...] = _ln_rows(e_ref[...], g_ref[...], b_ref[...])

    return pl.pallas_call(
        body,
        out_shape=[jax.ShapeDtypeStruct((_N, _D), _f32),
                   jax.ShapeDtypeStruct((1, _D), _f32)],
    )(pos, We1, emb, g0, b0)


def _edge_block_specs(n_edge_args):
    return [pl.BlockSpec((_BE, _D), lambda i: (i, 0)) for _ in range(n_edge_args)]


def _tc_edge_mlp_phi0(c0, n0, be1, We2, be2, We3, be3, eg, eb, hn0,
                     W1, b1, W2, b2):
    def body(c_ref, n_ref, bb, w2, b2_, w3, b3_, lg, lb, h0, pw1, pb1, pw2,
             pb2, e_ref, m_ref):
        h = _gelu(n_ref[...] - c_ref[...] + bb[...])
        h = _gelu(_dot(h, w2[...]) + b2_[...])
        e = _dot(h, w3[...]) + b3_[...]
        e = _ln_rows(e, lg[...], lb[...])
        e_ref[...] = e.astype(jnp.bfloat16)
        h0v = h0[...]
        u = jax.nn.silu(e + 2.0 * h0v)
        t = jax.nn.silu(_dot(u, pw1[...]) + pb1[...])
        t = _dot(t, pw2[...]) + pb2[...]
        m_ref[...] = t * h0v

    wspec = pl.BlockSpec((_D, _D), lambda i: (0, 0))
    bspec = pl.BlockSpec((1, _D), lambda i: (0, 0))
    return pl.pallas_call(
        body,
        grid=(_GE,),
        in_specs=_edge_block_specs(2) + [bspec, wspec, bspec, wspec, bspec,
                                         bspec, bspec, bspec, wspec, bspec,
                                         wspec, bspec],
        out_specs=[pl.BlockSpec((_BE, _D), lambda i: (i, 0)),
                   pl.BlockSpec((_BE, _D), lambda i: (i, 0))],
        out_shape=[jax.ShapeDtypeStruct((_EH, _D), jnp.bfloat16),
                   jax.ShapeDtypeStruct((_EH, _D), _f32)],
    )(c0, n0, be1, We2, be2, We3, be3, eg, eb, hn0, W1, b1, W2, b2)


def _tc_phi(e, c, nb, W1, b1, W2, b2):
    def body(e_ref, c_ref, nb_ref, w1, bb1, w2, bb2, out_ref):
        nbv = nb_ref[...]
        u = jax.nn.silu(e_ref[...].astype(_f32) + c_ref[...] + nbv)
        t = jax.nn.silu(_dot(u, w1[...]) + bb1[...])
        t = _dot(t, w2[...]) + bb2[...]
        out_ref[...] = t * nbv

    wspec = pl.BlockSpec((_D, _H), lambda i: (0, 0))
    bspec = pl.BlockSpec((1, _H), lambda i: (0, 0))
    wspec2 = pl.BlockSpec((_H, _D), lambda i: (0, 0))
    bspec2 = pl.BlockSpec((1, _D), lambda i: (0, 0))
    return pl.pallas_call(
        body,
        grid=(_GE,),
        in_specs=_edge_block_specs(3) + [wspec, bspec, wspec2, bspec2],
        out_specs=pl.BlockSpec((_BE, _D), lambda i: (i, 0)),
        out_shape=jax.ShapeDtypeStruct((_EH, _D), _f32),
    )(e, c, nb, W1, b1, W2, b2)


def _tc_update(node, hn, a00, a01, a10, a11, thW, thb, gn, bn):
    def body(n_ref, hn_ref, a00_ref, a01_ref, a10_ref, a11_ref, w_ref, b_ref,
             g_ref, lb_ref, out_ref, hnn_ref):
        agg = (a00_ref[...] + a01_ref[...]) + (a10_ref[...] + a11_ref[...])
        x = jax.nn.silu(hn_ref[...] + agg)
        nn = _dot(x, w_ref[...]) + b_ref[...] + n_ref[...]
        out_ref[...] = nn
        hnn_ref[...] = _ln_rows(nn, g_ref[...], lb_ref[...])

    return pl.pallas_call(
        body,
        out_shape=[jax.ShapeDtypeStruct((_N, _D), _f32),
                   jax.ShapeDtypeStruct((_N, _D), _f32)],
    )(node, hn, a00, a01, a10, a11, thW, thb, gn, bn)


def _tc_update_final(node, hn, a00, a01, a10, a11, thW, thb, Wf1, bf1, Wf2p,
                     bf2p):
    def body(n_ref, hn_ref, a00_ref, a01_ref, a10_ref, a11_ref, w_ref, b_ref,
             wf1, bff1, wf2, bff2, out_ref):
        agg = (a00_ref[...] + a01_ref[...]) + (a10_ref[...] + a11_ref[...])
        x = jax.nn.silu(hn_ref[...] + agg)
        nn = _dot(x, w_ref[...]) + b_ref[...] + n_ref[...]
        f = _gelu(_dot(nn, wf1[...]) + bff1[...])
        out_ref[...] = _dot(f, wf2[...]) + bff2[...]

    return pl.pallas_call(
        body,
        out_shape=jax.ShapeDtypeStruct((_N, _D), _f32),
    )(node, hn, a00, a01, a10, a11, thW, thb, Wf1, bf1, Wf2p, bf2p)


def kernel(pos, edge_index_list, We1, be1, We2, be2, We3, be3, eln_g, eln_b,
           node_emb, ln_g, ln_b, phiW1, phib1, phiW2, phib2, thW, thb,
           Wf1, bf1, Wf2, bf2):
    src = edge_index_list[0]
    dst = edge_index_list[1]
    g2, sct = _get_sc_kernels()

    r1 = lambda v: v.reshape(1, -1)

    posW, hn0 = _tc_prep(pos, We1, node_emb, r1(ln_g[0]), r1(ln_b[0]))
    eh = []
    aggs = []
    for h in (0, 1):
        c0, n0 = g2[h](posW, src, dst)
        e_h, msg = _tc_edge_mlp_phi0(c0, n0, r1(be1), We2, r1(be2), We3,
                                     r1(be3), r1(eln_g), r1(eln_b), hn0,
                                     phiW1[0], r1(phib1[0]), phiW2[0],
                                     r1(phib2[0]))
        eh.append(e_h)
        aggs.append(sct[h](msg, src))
    node0 = jnp.broadcast_to(node_emb, (_N, _D))
    hn0b = jnp.broadcast_to(hn0, (_N, _D))
    node, hn = _tc_update(node0, hn0b,
                          aggs[0][0, :_N], aggs[0][1, :_N],
                          aggs[1][0, :_N], aggs[1][1, :_N],
                          thW[0], r1(thb[0]), r1(ln_g[1]), r1(ln_b[1]))

    for l in (1, 2, 3):
        aggs = []
        for h in (0, 1):
            c, nb = g2[h](hn, src, dst)
            msg = _tc_phi(eh[h], c, nb, phiW1[l], r1(phib1[l]), phiW2[l],
                          r1(phib2[l]))
            aggs.append(sct[h](msg, src))
        if l < 3:
            node, hn = _tc_update(node, hn,
                                  aggs[0][0, :_N], aggs[0][1, :_N],
                                  aggs[1][0, :_N], aggs[1][1, :_N],
                                  thW[l], r1(thb[l]), r1(ln_g[l + 1]),
                                  r1(ln_b[l + 1]))

    Wf2p = jnp.pad(Wf2, ((0, 0), (0, _D - Wf2.shape[1])))
    bf2p = jnp.pad(bf2, ((0, _D - bf2.shape[0]),)).reshape(1, _D)
    fpad = _tc_update_final(node, hn,
                            aggs[0][0, :_N], aggs[0][1, :_N],
                            aggs[1][0, :_N], aggs[1][1, :_N],
                            thW[3], r1(thb[3]), Wf1, r1(bf1), Wf2p, bf2p)
    return fpad[:, :3]
